# baseline (speedup 1.0000x reference)
import functools

import jax
import jax.numpy as jnp
from jax import lax
from jax.experimental import pallas as pl
from jax.experimental.pallas import tpu as pltpu
from jax.experimental.pallas import tpu_sc as plsc

N = 100000
NHID = 128
NC = 2
NS = 16
NW = NC * NS
C = 128
FULL = N // C
CPW = FULL // NW
EXTRA = FULL - CPW * NW
TAIL_ROWS = N - FULL * C
TAILW = NW - 1
KMAX = CPW + 1

_mesh = plsc.VectorSubcoreMesh(core_axis_name="c", subcore_axis_name="s")


@functools.partial(
    pl.kernel,
    out_type=jax.ShapeDtypeStruct((N, NHID), jnp.float32),
    mesh=_mesh,
    scratch_types=[
        pltpu.VMEM((KMAX * C,), jnp.int32),
        pltpu.VMEM((C, NHID), jnp.float32),
        pltpu.VMEM((C, NHID), jnp.float32),
        pltpu.VMEM((C, NHID), jnp.float32),
        pltpu.VMEM((C, NHID), jnp.float32),
        pltpu.SemaphoreType.DMA,
        pltpu.SemaphoreType.DMA,
        pltpu.SemaphoreType.DMA,
        pltpu.SemaphoreType.DMA,
    ],
)
def _wl_pe(h_hbm, idx_hbm, w_hbm, out_hbm,
           idx_v, h0, h1, g0, g1, s0, s1, t0, t1):
    wid = lax.axis_index("s") * NC + lax.axis_index("c")
    b0 = jnp.where(
        wid < EXTRA, KMAX * wid, KMAX * EXTRA + CPW * (wid - EXTRA)
    ).astype(jnp.int32)

    @pl.when(wid < EXTRA)
    def _load_idx_25():
        src = pl.ds(pl.multiple_of(b0 * C, C), KMAX * C)
        pltpu.sync_copy(idx_hbm.at[src], idx_v)

    @pl.when(wid >= EXTRA)
    def _load_idx_24():
        src = pl.ds(pl.multiple_of(b0 * C, C), CPW * C)
        pltpu.sync_copy(idx_hbm.at[src], idx_v.at[pl.ds(0, CPW * C)])

    @pl.when(wid == TAILW)
    def _load_idx_tail():
        src = pl.ds(pl.multiple_of(FULL * C, C), TAIL_ROWS)
        pltpu.sync_copy(idx_hbm.at[src], idx_v.at[pl.ds(CPW * C, TAIL_ROWS)])

    hb, gb = (h0, h1), (g0, g1)
    sb, tb = (s0, s1), (t0, t1)

    def row0_of(k):
        return pl.multiple_of((b0 + k) * C, C)

    def start_in(k, b):
        r0 = row0_of(k)
        pltpu.async_copy(h_hbm.at[pl.ds(r0, C)], hb[b], sb[b])
        idx_ref = idx_v.at[pl.ds(k * C, C)]
        pltpu.async_copy(w_hbm.at[idx_ref], gb[b], sb[b])

    def wait_in(b):
        pltpu.make_async_copy(h_hbm.at[pl.ds(0, C)], hb[b], sb[b]).wait()
        pltpu.make_async_copy(h_hbm.at[pl.ds(0, C)], gb[b], sb[b]).wait()

    def add(b, rows=C):
        def add_row(r, carry):
            for j in range(NHID // 16):
                sl = pl.ds(j * 16, 16)
                plsc.addupdate(gb[b].at[r, sl], hb[b][r, sl])
            return carry

        lax.fori_loop(0, rows, add_row, 0)

    def start_out(k, b):
        pltpu.async_copy(gb[b], out_hbm.at[pl.ds(row0_of(k), C)], tb[b])

    def wait_out(b):
        pltpu.make_async_copy(h_hbm.at[pl.ds(0, C)], gb[b], tb[b]).wait()

    start_in(0, 0)
    wait_in(0)
    add(0)
    start_out(0, 0)
    start_in(1, 1)

    def pair(i, carry):
        k = 2 * i + 1
        wait_in(1)
        add(1)
        start_out(k, 1)
        wait_out(0)
        start_in(k + 1, 0)
        wait_in(0)
        add(0)
        start_out(k + 1, 0)
        wait_out(1)
        start_in(k + 2, 1)
        return carry

    lax.fori_loop(0, (CPW - 2) // 2, pair, 0)

    wait_in(1)
    add(1)
    start_out(CPW - 1, 1)
    wait_out(0)

    @pl.when(wid < EXTRA)
    def _extra():
        start_in(CPW, 0)
        wait_in(0)
        add(0)
        start_out(CPW, 0)
        wait_out(0)

    @pl.when(wid == TAILW)
    def _tail():
        r0 = pl.multiple_of(FULL * C, C)
        t = pl.ds(0, TAIL_ROWS)
        pltpu.sync_copy(h_hbm.at[pl.ds(r0, TAIL_ROWS)], h0.at[t])
        idx_ref = idx_v.at[pl.ds(CPW * C, TAIL_ROWS)]
        pltpu.async_copy(w_hbm.at[idx_ref], g0.at[t], s0).wait()
        add(0, rows=TAIL_ROWS)
        pltpu.sync_copy(g0.at[t], out_hbm.at[pl.ds(r0, TAIL_ROWS)])

    wait_out(1)


def kernel(h, precomputed_eigenvectors, proj_weight):
    idx = precomputed_eigenvectors.astype(jnp.int32)
    return _wl_pe(h, idx, proj_weight)


# speedup vs baseline: 1.4574x; 1.4574x over previous
import functools

import jax
import jax.numpy as jnp
from jax import lax
from jax.experimental import pallas as pl
from jax.experimental.pallas import tpu as pltpu
from jax.experimental.pallas import tpu_sc as plsc

N = 100000
NHID = 128
NC = 2
NS = 16
NW = NC * NS
C = 128
FULL = N // C
CPW = FULL // NW
EXTRA = FULL - CPW * NW
TAIL_ROWS = N - FULL * C
TAILW = NW - 1
KMAX = CPW + 1

_mesh = plsc.VectorSubcoreMesh(core_axis_name="c", subcore_axis_name="s")


@functools.partial(
    pl.kernel,
    out_type=jax.ShapeDtypeStruct((N, NHID), jnp.float32),
    mesh=_mesh,
    scratch_types=[
        pltpu.VMEM((KMAX * C,), jnp.int32),
        pltpu.VMEM((C, NHID), jnp.float32),
        pltpu.VMEM((C, NHID), jnp.float32),
        pltpu.VMEM((C, NHID), jnp.float32),
        pltpu.VMEM((C, NHID), jnp.float32),
        pltpu.VMEM((C, NHID), jnp.float32),
        pltpu.VMEM((C, NHID), jnp.float32),
        pltpu.SemaphoreType.DMA,
        pltpu.SemaphoreType.DMA,
        pltpu.SemaphoreType.DMA,
        pltpu.SemaphoreType.DMA,
    ],
)
def _wl_pe(h_hbm, idx_hbm, w_hbm, out_hbm,
           idx_v, h0, h1, g0, g1, o0, o1, s0, s1, t0, t1):
    wid = lax.axis_index("s") * NC + lax.axis_index("c")
    b0 = jnp.where(
        wid < EXTRA, KMAX * wid, KMAX * EXTRA + CPW * (wid - EXTRA)
    ).astype(jnp.int32)

    @pl.when(wid < EXTRA)
    def _load_idx_25():
        src = pl.ds(pl.multiple_of(b0 * C, C), KMAX * C)
        pltpu.sync_copy(idx_hbm.at[src], idx_v)

    @pl.when(wid >= EXTRA)
    def _load_idx_24():
        src = pl.ds(pl.multiple_of(b0 * C, C), CPW * C)
        pltpu.sync_copy(idx_hbm.at[src], idx_v.at[pl.ds(0, CPW * C)])

    @pl.when(wid == TAILW)
    def _load_idx_tail():
        src = pl.ds(pl.multiple_of(FULL * C, C), TAIL_ROWS)
        pltpu.sync_copy(idx_hbm.at[src], idx_v.at[pl.ds(CPW * C, TAIL_ROWS)])

    hb, gb, ob = (h0, h1), (g0, g1), (o0, o1)
    sb, tb = (s0, s1), (t0, t1)

    def row0_of(k):
        return pl.multiple_of((b0 + k) * C, C)

    def start_in(k, b):
        r0 = row0_of(k)
        pltpu.async_copy(h_hbm.at[pl.ds(r0, C)], hb[b], sb[b])
        idx_ref = idx_v.at[pl.ds(k * C, C)]
        pltpu.async_copy(w_hbm.at[idx_ref], gb[b], sb[b])

    def wait_in(b):
        pltpu.make_async_copy(h_hbm.at[pl.ds(0, C)], hb[b], sb[b]).wait()
        pltpu.make_async_copy(h_hbm.at[pl.ds(0, C)], gb[b], sb[b]).wait()

    def add3(b):
        def add_row(r, carry):
            for j in range(NHID // 16):
                sl = pl.ds(j * 16, 16)
                ob[b][r, sl] = hb[b][r, sl] + gb[b][r, sl]
            return carry

        lax.fori_loop(0, C, add_row, 0)

    def start_out(k, b):
        pltpu.async_copy(ob[b], out_hbm.at[pl.ds(row0_of(k), C)], tb[b])

    def wait_out(b):
        pltpu.make_async_copy(h_hbm.at[pl.ds(0, C)], ob[b], tb[b]).wait()

    start_in(0, 0)
    start_in(1, 1)
    wait_in(0)
    add3(0)
    start_out(0, 0)
    start_in(2, 0)
    wait_in(1)
    add3(1)
    start_out(1, 1)

    def pair(i, carry):
        k = 2 * i + 2
        start_in(k + 1, 1)
        wait_in(0)
        wait_out(0)
        add3(0)
        start_out(k, 0)
        start_in(k + 2, 0)
        wait_in(1)
        wait_out(1)
        add3(1)
        start_out(k + 1, 1)
        return carry

    lax.fori_loop(0, (CPW - 4) // 2, pair, 0)

    start_in(CPW - 1, 1)
    wait_in(0)
    wait_out(0)
    add3(0)
    start_out(CPW - 2, 0)

    @pl.when(wid < EXTRA)
    def _start_extra():
        start_in(CPW, 0)

    wait_in(1)
    wait_out(1)
    add3(1)
    start_out(CPW - 1, 1)

    @pl.when(wid < EXTRA)
    def _finish_extra():
        wait_in(0)
        wait_out(0)
        add3(0)
        start_out(CPW, 0)

    @pl.when(wid == TAILW)
    def _tail():
        r0 = pl.multiple_of(FULL * C, C)
        t = pl.ds(0, TAIL_ROWS)
        pltpu.sync_copy(h_hbm.at[pl.ds(r0, TAIL_ROWS)], h0.at[t])
        idx_ref = idx_v.at[pl.ds(CPW * C, TAIL_ROWS)]
        pltpu.async_copy(w_hbm.at[idx_ref], g0.at[t], s0).wait()

        def add_row(r, carry):
            for j in range(NHID // 16):
                sl = pl.ds(j * 16, 16)
                plsc.addupdate(g0.at[r, sl], h0[r, sl])
            return carry

        lax.fori_loop(0, TAIL_ROWS, add_row, 0)
        pltpu.sync_copy(g0.at[t], out_hbm.at[pl.ds(r0, TAIL_ROWS)])

    wait_out(0)
    wait_out(1)


def kernel(h, precomputed_eigenvectors, proj_weight):
    idx = precomputed_eigenvectors.astype(jnp.int32)
    return _wl_pe(h, idx, proj_weight)


# speedup vs baseline: 1.4612x; 1.0027x over previous
import functools

import jax
import jax.numpy as jnp
from jax import lax
from jax.experimental import pallas as pl
from jax.experimental.pallas import tpu as pltpu
from jax.experimental.pallas import tpu_sc as plsc

N = 100000
NHID = 128
NC = 2
NS = 16
NW = NC * NS
C = 128
FULL = N // C
CPW = FULL // NW
EXTRA = FULL - CPW * NW
TAIL_ROWS = N - FULL * C
TAILW = NW - 1
KMAX = CPW + 1

_mesh = plsc.VectorSubcoreMesh(core_axis_name="c", subcore_axis_name="s")


@functools.partial(
    pl.kernel,
    out_type=jax.ShapeDtypeStruct((N, NHID), jnp.float32),
    mesh=_mesh,
    scratch_types=[
        pltpu.VMEM((KMAX * C,), jnp.int32),
        pltpu.VMEM((C, NHID), jnp.float32),
        pltpu.VMEM((C, NHID), jnp.float32),
        pltpu.VMEM((C, NHID), jnp.float32),
        pltpu.VMEM((C, NHID), jnp.float32),
        pltpu.VMEM((C, NHID), jnp.float32),
        pltpu.VMEM((C, NHID), jnp.float32),
        pltpu.SemaphoreType.DMA,
        pltpu.SemaphoreType.DMA,
        pltpu.SemaphoreType.DMA,
        pltpu.SemaphoreType.DMA,
        pltpu.SemaphoreType.DMA,
        pltpu.SemaphoreType.DMA,
    ],
)
def _wl_pe(h_hbm, idx_hbm, w_hbm, out_hbm,
           idx_v, r0b, r1b, r2b, g0, g1, g2, s0, s1, s2, t0, t1, t2):
    wid = lax.axis_index("s") * NC + lax.axis_index("c")
    b0 = jnp.where(
        wid < EXTRA, KMAX * wid, KMAX * EXTRA + CPW * (wid - EXTRA)
    ).astype(jnp.int32)

    @pl.when(wid < EXTRA)
    def _load_idx_25():
        src = pl.ds(pl.multiple_of(b0 * C, C), KMAX * C)
        pltpu.sync_copy(idx_hbm.at[src], idx_v)

    @pl.when(wid >= EXTRA)
    def _load_idx_24():
        src = pl.ds(pl.multiple_of(b0 * C, C), CPW * C)
        pltpu.sync_copy(idx_hbm.at[src], idx_v.at[pl.ds(0, CPW * C)])

    @pl.when(wid == TAILW)
    def _load_idx_tail():
        src = pl.ds(pl.multiple_of(FULL * C, C), TAIL_ROWS)
        pltpu.sync_copy(idx_hbm.at[src], idx_v.at[pl.ds(CPW * C, TAIL_ROWS)])

    rb, gb = (r0b, r1b, r2b), (g0, g1, g2)
    sb, tb = (s0, s1, s2), (t0, t1, t2)

    def row0_of(k):
        return pl.multiple_of((b0 + k) * C, C)

    def start_in(k, b):
        r0 = row0_of(k)
        pltpu.async_copy(h_hbm.at[pl.ds(r0, C)], rb[b], sb[b])
        idx_ref = idx_v.at[pl.ds(k * C, C)]
        pltpu.async_copy(w_hbm.at[idx_ref], gb[b], sb[b])

    def wait_in(b):
        pltpu.make_async_copy(h_hbm.at[pl.ds(0, C)], rb[b], sb[b]).wait()
        pltpu.make_async_copy(h_hbm.at[pl.ds(0, C)], gb[b], sb[b]).wait()

    def add(b, rows=C):
        def add_row(r, carry):
            for j in range(NHID // 16):
                sl = pl.ds(j * 16, 16)
                plsc.addupdate(rb[b].at[r, sl], gb[b][r, sl])
            return carry

        lax.fori_loop(0, rows, add_row, 0)

    def start_out(k, b):
        pltpu.async_copy(rb[b], out_hbm.at[pl.ds(row0_of(k), C)], tb[b])

    def wait_out(b):
        pltpu.make_async_copy(h_hbm.at[pl.ds(0, C)], rb[b], tb[b]).wait()

    start_in(0, 0)
    start_in(1, 1)
    wait_in(0)
    add(0)
    start_out(0, 0)
    start_in(2, 2)

    def step(k, b):
        wait_in(b)
        add(b)
        start_out(k, b)
        b2 = (b + 2) % 3
        wait_out(b2)
        start_in(k + 2, b2)

    def triple(i, carry):
        k = 3 * i + 1
        step(k, 1)
        step(k + 1, 2)
        step(k + 2, 0)
        return carry

    lax.fori_loop(0, (CPW - 3) // 3, triple, 0)

    wait_in(1)
    add(1)
    start_out(CPW - 2, 1)
    wait_out(0)

    @pl.when(wid < EXTRA)
    def _start_extra():
        start_in(CPW, 0)

    wait_in(2)
    add(2)
    start_out(CPW - 1, 2)

    @pl.when(wid < EXTRA)
    def _finish_extra():
        wait_in(0)
        add(0)
        start_out(CPW, 0)
        wait_out(0)

    @pl.when(wid == TAILW)
    def _tail():
        r0 = pl.multiple_of(FULL * C, C)
        t = pl.ds(0, TAIL_ROWS)
        pltpu.sync_copy(h_hbm.at[pl.ds(r0, TAIL_ROWS)], r0b.at[t])
        idx_ref = idx_v.at[pl.ds(CPW * C, TAIL_ROWS)]
        pltpu.async_copy(w_hbm.at[idx_ref], g0.at[t], s0).wait()
        add(0, rows=TAIL_ROWS)
        pltpu.sync_copy(r0b.at[t], out_hbm.at[pl.ds(r0, TAIL_ROWS)])

    wait_out(1)
    wait_out(2)


def kernel(h, precomputed_eigenvectors, proj_weight):
    idx = precomputed_eigenvectors.astype(jnp.int32)
    return _wl_pe(h, idx, proj_weight)
